# Initial kernel scaffold; baseline (speedup 1.0000x reference)
import jax
import jax.numpy as jnp
from jax.experimental import pallas as pl


def kernel(h, edge_index, efeat, params):
    raise NotImplementedError("write your pallas kernel here")


# speedup vs baseline: 1.5865x; 1.5865x over previous
import functools

import jax
import jax.numpy as jnp
from jax import lax
from jax.experimental import pallas as pl
from jax.experimental.pallas import tpu as pltpu
from jax.experimental.pallas import tpu_sc as plsc

_N = 10000
_E = 160000
_D_IN = 128
_D_EDGE = 16
_HID = 64
_OUT = 64
_BN_EPS = 1e-5

_CHUNK = 128
_NCHUNKS = _E // _CHUNK
_NW = 32
_CHUNKS_PER_W = -(-_NCHUNKS // _NW)
_NPAD = 10240
_ROWS_PER_SUB = _NPAD // 16

_BE = 2000
_NBLK = _E // _BE


def _sc_mesh():
    return plsc.VectorSubcoreMesh(core_axis_name="c", subcore_axis_name="s")


def _worker_id():
    return lax.axis_index("s") * 2 + lax.axis_index("c")


def _gather_body(idx_hbm, table_hbm, out_hbm, idx_v, rows_v, sem):
    wid = _worker_id()

    def chunk(k, carry):
        c = wid + k * _NW

        @pl.when(c < _NCHUNKS)
        def _():
            start = pl.multiple_of(c * _CHUNK, 8)
            pltpu.sync_copy(idx_hbm.at[pl.ds(start, _CHUNK)], idx_v)
            pltpu.async_copy(table_hbm.at[idx_v], rows_v, sem).wait()
            pltpu.sync_copy(rows_v, out_hbm.at[pl.ds(start, _CHUNK)])

        return carry

    lax.fori_loop(0, _CHUNKS_PER_W, chunk, 0)


def _sc_gather(table, idx, width):
    return pl.kernel(
        _gather_body,
        out_type=jax.ShapeDtypeStruct((_E, width), jnp.float32),
        mesh=_sc_mesh(),
        scratch_types=[
            pltpu.VMEM((_CHUNK,), jnp.int32),
            pltpu.VMEM((_CHUNK, width), jnp.float32),
            pltpu.SemaphoreType.DMA,
        ],
        compiler_params=pltpu.CompilerParams(use_tc_tiling_on_sc=False),
    )(idx, table)


def _scatter_body(dst_hbm, msg_hbm, zeros_hbm, out_hbm, idx_v, msg_v, agg_sh):
    cid = lax.axis_index("c")
    sid = lax.axis_index("s")
    wid = _worker_id()

    row0 = pl.multiple_of(sid * _ROWS_PER_SUB, 8)
    pltpu.sync_copy(zeros_hbm, agg_sh.at[pl.ds(row0, _ROWS_PER_SUB)])
    plsc.subcore_barrier()

    def chunk(k, carry):
        c = wid + k * _NW

        @pl.when(c < _NCHUNKS)
        def _():
            start = pl.multiple_of(c * _CHUNK, 8)
            pltpu.sync_copy(dst_hbm.at[pl.ds(start, _CHUNK)], idx_v)
            pltpu.sync_copy(msg_hbm.at[pl.ds(start, _CHUNK)], msg_v)
            pltpu.sync_copy(msg_v, agg_sh.at[idx_v], add=True)

        return carry

    lax.fori_loop(0, _CHUNKS_PER_W, chunk, 0)
    plsc.subcore_barrier()
    out0 = pl.multiple_of(cid * _NPAD + row0, 8)
    pltpu.sync_copy(agg_sh.at[pl.ds(row0, _ROWS_PER_SUB)],
                    out_hbm.at[pl.ds(out0, _ROWS_PER_SUB)])


def _sc_scatter_add(msg, dst):
    zeros = jnp.zeros((_ROWS_PER_SUB, _HID), jnp.float32)
    return pl.kernel(
        _scatter_body,
        out_type=jax.ShapeDtypeStruct((2 * _NPAD, _HID), jnp.float32),
        mesh=_sc_mesh(),
        scratch_types=[
            pltpu.VMEM((_CHUNK,), jnp.int32),
            pltpu.VMEM((_CHUNK, _HID), jnp.float32),
            pltpu.VMEM_SHARED((_NPAD, _HID), jnp.float32),
        ],
        compiler_params=pltpu.CompilerParams(use_tc_tiling_on_sc=False),
    )(dst, msg, zeros)


def _msg_body(hs_ref, ef_ref, a_ref, b_ref, eps_ref, out_ref):
    hs = hs_ref[...]
    acc = jnp.dot(hs, b_ref[...], preferred_element_type=jnp.float32)
    ef = ef_ref[...]
    for f in range(_D_EDGE):
        acc += ef[:, f:f + 1] * jnp.dot(hs, a_ref[f],
                                        preferred_element_type=jnp.float32)
    out_ref[...] = (1.0 + eps_ref[0, 0]) * acc


def _tc_msg(h_src, efeat, a, bmat, eps):
    return pl.pallas_call(
        _msg_body,
        grid=(_NBLK,),
        in_specs=[
            pl.BlockSpec((_BE, _D_IN), lambda i: (i, 0)),
            pl.BlockSpec((_BE, _D_EDGE), lambda i: (i, 0)),
            pl.BlockSpec((_D_EDGE, _D_IN, _HID), lambda i: (0, 0, 0)),
            pl.BlockSpec((_D_IN, _HID), lambda i: (0, 0)),
            pl.BlockSpec((1, 1), lambda i: (0, 0)),
        ],
        out_specs=pl.BlockSpec((_BE, _HID), lambda i: (i, 0)),
        out_shape=jax.ShapeDtypeStruct((_E, _HID), jnp.float32),
    )(h_src, efeat, a, bmat, eps)


def _bn_cols(x, g, b):
    mu = jnp.mean(x, axis=0, keepdims=True)
    xc = x - mu
    var = jnp.mean(xc * xc, axis=0, keepdims=True)
    return xc * lax.rsqrt(var + _BN_EPS) * g + b


def _leaky(x):
    return jnp.where(x >= 0, x, 0.01 * x)


def _node_body(agg_ref, w1_ref, b1_ref, g1_ref, be1_ref, w2_ref, b2_ref,
               g_ref, be_ref, wo1_ref, bo1_ref, pw_ref, pb_ref,
               v_ref, hp_ref):
    agg = agg_ref[0:_N, :] + agg_ref[_NPAD:_NPAD + _N, :]
    x = jnp.dot(agg, w1_ref[...], preferred_element_type=jnp.float32) + b1_ref[...]
    hr = jnp.maximum(_bn_cols(x, g1_ref[...], be1_ref[...]), 0.0)
    x2 = jnp.dot(hr, w2_ref[...], preferred_element_type=jnp.float32) + b2_ref[...]
    h1 = _leaky(_bn_cols(x2, g_ref[...], be_ref[...]))
    v_ref[...] = jnp.dot(h1, wo1_ref[...], preferred_element_type=jnp.float32) + bo1_ref[...]
    hp_ref[...] = jnp.dot(h1, pw_ref[...], preferred_element_type=jnp.float32) + pb_ref[...]


def _tc_node(agg2, p):
    full = lambda s: pl.BlockSpec(s, lambda: tuple(0 for _ in s))
    return pl.pallas_call(
        _node_body,
        in_specs=[
            full((2 * _NPAD, _HID)),
            full((_HID, _HID)), full((1, _HID)), full((1, _HID)), full((1, _HID)),
            full((_HID, _HID)), full((1, _HID)), full((1, _HID)), full((1, _HID)),
            full((_HID, _OUT)), full((1, _OUT)),
            full((_HID, _D_EDGE)), full((1, _D_EDGE)),
        ],
        out_specs=[full((_N, _OUT)), full((_N, _D_EDGE))],
        out_shape=[
            jax.ShapeDtypeStruct((_N, _OUT), jnp.float32),
            jax.ShapeDtypeStruct((_N, _D_EDGE), jnp.float32),
        ],
    )(agg2,
      p['nc_w1'], p['nc_b1'].reshape(1, -1), p['nc_g1'].reshape(1, -1),
      p['nc_be1'].reshape(1, -1),
      p['nc_w2'], p['nc_b2'].reshape(1, -1), p['nc_g'].reshape(1, -1),
      p['nc_be'].reshape(1, -1),
      p['wo1'], p['bo1'].reshape(1, -1),
      p['ec_pw'], p['ec_pb'].reshape(1, -1))


def _hein_body(ef_ref, hs_ref, hd_ref, eps_ref, out_ref, s1_ref, m1_ref):
    i = pl.program_id(0)
    he = (1.0 + eps_ref[0, 0]) * ef_ref[...] + hs_ref[...] + hd_ref[...]
    out_ref[...] = he
    s = jnp.sum(he, axis=0, keepdims=True)
    m = lax.dot_general(he, he, (((0,), (0,)), ((), ())),
                        preferred_element_type=jnp.float32)

    @pl.when(i == 0)
    def _():
        s1_ref[...] = s
        m1_ref[...] = m

    @pl.when(i > 0)
    def _():
        s1_ref[...] += s
        m1_ref[...] += m


def _tc_hein(efeat, hp_src, hp_dst, eps):
    blk = lambda w: pl.BlockSpec((_BE, w), lambda i: (i, 0))
    return pl.pallas_call(
        _hein_body,
        grid=(_NBLK,),
        in_specs=[blk(_D_EDGE), blk(_D_EDGE), blk(_D_EDGE),
                  pl.BlockSpec((1, 1), lambda i: (0, 0))],
        out_specs=[
            pl.BlockSpec((_BE, _D_EDGE), lambda i: (i, 0)),
            pl.BlockSpec((1, _D_EDGE), lambda i: (0, 0)),
            pl.BlockSpec((_D_EDGE, _D_EDGE), lambda i: (0, 0)),
        ],
        out_shape=[
            jax.ShapeDtypeStruct((_E, _D_EDGE), jnp.float32),
            jax.ShapeDtypeStruct((1, _D_EDGE), jnp.float32),
            jax.ShapeDtypeStruct((_D_EDGE, _D_EDGE), jnp.float32),
        ],
    )(efeat, hp_src, hp_dst, eps)


def _bn_stats(s, m, w, b):
    mean_y = s / _E
    mw = jnp.dot(mean_y, w, preferred_element_type=jnp.float32)
    mu = mw + b
    diag = jnp.sum(w * jnp.dot(m, w, preferred_element_type=jnp.float32),
                   axis=0, keepdims=True)
    ex2 = diag / _E + 2.0 * b * mw + b * b
    return mu, ex2 - mu * mu


def _stats2_body(he_ref, s1_ref, m1_ref, w1_ref, b1_ref, g1_ref, be1_ref,
                 s2_ref, m2_ref):
    i = pl.program_id(0)
    w1 = w1_ref[...]
    b1 = b1_ref[...]
    mu1, var1 = _bn_stats(s1_ref[...], m1_ref[...], w1, b1)
    x = jnp.dot(he_ref[...], w1, preferred_element_type=jnp.float32) + b1
    xn = (x - mu1) * lax.rsqrt(var1 + _BN_EPS) * g1_ref[...] + be1_ref[...]
    hr = jnp.maximum(xn, 0.0)
    s = jnp.sum(hr, axis=0, keepdims=True)
    m = lax.dot_general(hr, hr, (((0,), (0,)), ((), ())),
                        preferred_element_type=jnp.float32)

    @pl.when(i == 0)
    def _():
        s2_ref[...] = s
        m2_ref[...] = m

    @pl.when(i > 0)
    def _():
        s2_ref[...] += s
        m2_ref[...] += m


def _tc_stats2(he_in, s1, m1, p):
    full = lambda s: pl.BlockSpec(s, lambda i: tuple(0 for _ in s))
    return pl.pallas_call(
        _stats2_body,
        grid=(_NBLK,),
        in_specs=[
            pl.BlockSpec((_BE, _D_EDGE), lambda i: (i, 0)),
            full((1, _D_EDGE)), full((_D_EDGE, _D_EDGE)),
            full((_D_EDGE, _HID)), full((1, _HID)), full((1, _HID)),
            full((1, _HID)),
        ],
        out_specs=[full((1, _HID)), full((_HID, _HID))],
        out_shape=[
            jax.ShapeDtypeStruct((1, _HID), jnp.float32),
            jax.ShapeDtypeStruct((_HID, _HID), jnp.float32),
        ],
    )(he_in, s1, m1, p['ec_w1'], p['ec_b1'].reshape(1, -1),
      p['ec_g1'].reshape(1, -1), p['ec_be1'].reshape(1, -1))


def _edge_out_body(he_ref, s1_ref, m1_ref, s2_ref, m2_ref, w1_ref, b1_ref,
                   g1_ref, be1_ref, w2_ref, b2_ref, g_ref, be_ref,
                   wo2_ref, bo2_ref, out_ref):
    w1 = w1_ref[...]
    b1 = b1_ref[...]
    w2 = w2_ref[...]
    b2 = b2_ref[...]
    mu1, var1 = _bn_stats(s1_ref[...], m1_ref[...], w1, b1)
    mu2, var2 = _bn_stats(s2_ref[...], m2_ref[...], w2, b2)
    x = jnp.dot(he_ref[...], w1, preferred_element_type=jnp.float32) + b1
    xn = (x - mu1) * lax.rsqrt(var1 + _BN_EPS) * g1_ref[...] + be1_ref[...]
    hr = jnp.maximum(xn, 0.0)
    x2 = jnp.dot(hr, w2, preferred_element_type=jnp.float32) + b2
    xn2 = (x2 - mu2) * lax.rsqrt(var2 + _BN_EPS) * g_ref[...] + be_ref[...]
    he = _leaky(xn2)
    out_ref[...] = jnp.dot(he, wo2_ref[...],
                           preferred_element_type=jnp.float32) + bo2_ref[...]


def _tc_edge_out(he_in, s1, m1, s2, m2, p):
    full = lambda s: pl.BlockSpec(s, lambda i: tuple(0 for _ in s))
    return pl.pallas_call(
        _edge_out_body,
        grid=(_NBLK,),
        in_specs=[
            pl.BlockSpec((_BE, _D_EDGE), lambda i: (i, 0)),
            full((1, _D_EDGE)), full((_D_EDGE, _D_EDGE)),
            full((1, _HID)), full((_HID, _HID)),
            full((_D_EDGE, _HID)), full((1, _HID)), full((1, _HID)),
            full((1, _HID)),
            full((_HID, _HID)), full((1, _HID)), full((1, _HID)),
            full((1, _HID)),
            full((_HID, _OUT)), full((1, _OUT)),
        ],
        out_specs=pl.BlockSpec((_BE, _OUT), lambda i: (i, 0)),
        out_shape=jax.ShapeDtypeStruct((_E, _OUT), jnp.float32),
    )(he_in, s1, m1, s2, m2,
      p['ec_w1'], p['ec_b1'].reshape(1, -1), p['ec_g1'].reshape(1, -1),
      p['ec_be1'].reshape(1, -1),
      p['ec_w2'], p['ec_b2'].reshape(1, -1), p['ec_g'].reshape(1, -1),
      p['ec_be'].reshape(1, -1),
      p['wo2'], p['bo2'].reshape(1, -1))


def kernel(h, edge_index, efeat, params):
    p = params
    src = edge_index[0]
    dst = edge_index[1]
    nc_eps = p['nc_eps'].reshape(1, 1)
    ec_eps = p['ec_eps'].reshape(1, 1)

    h_src = _sc_gather(h, src, _D_IN)
    msg = _tc_msg(h_src, efeat, p['A'], p['Bmat'], nc_eps)
    agg2 = _sc_scatter_add(msg, dst)
    v_emb, hp = _tc_node(agg2, p)
    hp_src = _sc_gather(hp, src, _D_EDGE)
    hp_dst = _sc_gather(hp, dst, _D_EDGE)
    he_in, s1, m1 = _tc_hein(efeat, hp_src, hp_dst, ec_eps)
    s2, m2 = _tc_stats2(he_in, s1, m1, p)
    e_emb = _tc_edge_out(he_in, s1, m1, s2, m2, p)
    return (v_emb, e_emb)
